# Initial kernel scaffold; baseline (speedup 1.0000x reference)
import jax
import jax.numpy as jnp
from jax.experimental import pallas as pl


def kernel(node_feats, edge_index, W_enc, b_enc, W_conv, b_conv, bn_gamma, bn_beta, W_out, b_out):
    raise NotImplementedError("write your pallas kernel here")


# speedup vs baseline: 8.2692x; 8.2692x over previous
import functools

import jax
import jax.numpy as jnp
from jax import lax
from jax.experimental import pallas as pl
from jax.experimental.pallas import tpu as pltpu
from jax.experimental.pallas import tpu_sc as plsc

N = 10000
E = 320000
HID = 128
OUT_DIM = 40
NUM_LAYERS = 7
EPS = 1e-7
BN_EPS = 1e-5

NT = 16
EB = 128
NB = -(-E // (NT * EB))
E_PAD = NT * NB * EB
ACC_ROWS = 10240
ZR = ACC_ROWS // NT
RB = 2000


def _sc_aggregate(tg, tp, src_p, dst_p, zeros_blk):
    mesh = plsc.VectorSubcoreMesh(core_axis_name="c", subcore_axis_name="s")

    @functools.partial(
        pl.kernel,
        out_type=(jax.ShapeDtypeStruct((ACC_ROWS, HID), jnp.float32),
                  jax.ShapeDtypeStruct((ACC_ROWS, HID), jnp.float32)),
        mesh=mesh,
        scratch_types=[
            pltpu.VMEM((EB,), jnp.int32),
            pltpu.VMEM((EB,), jnp.int32),
            pltpu.VMEM((EB, HID), jnp.float32),
            pltpu.VMEM_SHARED((ACC_ROWS, HID), jnp.float32),
        ],
    )
    def k(tg_hbm, tp_hbm, src_hbm, dst_hbm, z_hbm, den_out, num_out,
          sidx_v, didx_v, rows_v, acc):
        c = lax.axis_index("c")
        s = lax.axis_index("s")

        pltpu.sync_copy(z_hbm, acc.at[pl.ds(s * ZR, ZR)])
        plsc.subcore_barrier()

        def run(t_hbm):
            def body(j, carry):
                base = s * (NB * EB) + j * EB
                pltpu.sync_copy(src_hbm.at[pl.ds(base, EB)], sidx_v)
                pltpu.sync_copy(dst_hbm.at[pl.ds(base, EB)], didx_v)
                pltpu.sync_copy(t_hbm.at[sidx_v], rows_v)
                pltpu.sync_copy(rows_v, acc.at[didx_v], add=True)
                return carry
            lax.fori_loop(0, NB, body, 0)

        @pl.when(c == 0)
        def _():
            run(tg_hbm)

        @pl.when(c == 1)
        def _():
            run(tp_hbm)

        plsc.subcore_barrier()

        @pl.when(c == 0)
        def _():
            pltpu.sync_copy(acc.at[pl.ds(s * ZR, ZR)],
                            den_out.at[pl.ds(s * ZR, ZR)])

        @pl.when(c == 1)
        def _():
            pltpu.sync_copy(acc.at[pl.ds(s * ZR, ZR)],
                            num_out.at[pl.ds(s * ZR, ZR)])

    den, num = k(tg, tp, src_p, dst_p, zeros_blk)
    return den[:N], num[:N]


def _enc_body(x_ref, w_ref, b_ref, o_ref):
    o_ref[...] = jnp.dot(x_ref[...], w_ref[...],
                         preferred_element_type=jnp.float32) + b_ref[...]


def _encoder(x, w, b):
    return pl.pallas_call(
        _enc_body,
        grid=(N // RB,),
        in_specs=[pl.BlockSpec((RB, HID), lambda i: (i, 0)),
                  pl.BlockSpec((HID, HID), lambda i: (0, 0)),
                  pl.BlockSpec((1, HID), lambda i: (0, 0))],
        out_specs=pl.BlockSpec((RB, HID), lambda i: (i, 0)),
        out_shape=jax.ShapeDtypeStruct((N, HID), jnp.float32),
    )(x, w, b)


def _pre_body(h_ref, g_ref, b_ref, hn_ref, tg_ref, tp_ref):
    h = h_ref[...]
    mean = jnp.mean(h, axis=0, keepdims=True)
    d = h - mean
    var = jnp.mean(d * d, axis=0, keepdims=True)
    hn = d * lax.rsqrt(var + BN_EPS) * g_ref[...] + b_ref[...]
    hn = jnp.maximum(hn, 0.0)
    g = hn + EPS
    cmax = jnp.max(g, axis=0, keepdims=True)
    tg = jnp.exp(g - cmax)
    hn_ref[...] = hn
    tg_ref[...] = tg
    tp_ref[...] = g * tg


def _pre(h, gamma, beta):
    spec = pl.BlockSpec((N, HID), lambda: (0, 0))
    vspec = pl.BlockSpec((1, HID), lambda: (0, 0))
    return pl.pallas_call(
        _pre_body,
        in_specs=[spec, vspec, vspec],
        out_specs=(spec, spec, spec),
        out_shape=(jax.ShapeDtypeStruct((N, HID), jnp.float32),) * 3,
    )(h, gamma, beta)


def _post_body(hn_ref, num_ref, den_ref, h_ref, w_ref, b_ref, o_ref):
    msg = num_ref[...] / (den_ref[...] + 1e-16)
    l2 = jnp.sqrt(jnp.sum(msg * msg, axis=1, keepdims=True))
    hn = hn_ref[...]
    fn = jnp.sqrt(jnp.sum(hn * hn, axis=1, keepdims=True))
    msg = msg / jnp.maximum(l2, 1e-12) * fn
    feats = hn + msg
    o_ref[...] = h_ref[...] + jnp.dot(feats, w_ref[...],
                                      preferred_element_type=jnp.float32) + b_ref[...]


def _post(hn, num, den, h, w, b):
    rspec = pl.BlockSpec((RB, HID), lambda i: (i, 0))
    return pl.pallas_call(
        _post_body,
        grid=(N // RB,),
        in_specs=[rspec, rspec, rspec, rspec,
                  pl.BlockSpec((HID, HID), lambda i: (0, 0)),
                  pl.BlockSpec((1, HID), lambda i: (0, 0))],
        out_specs=rspec,
        out_shape=jax.ShapeDtypeStruct((N, HID), jnp.float32),
    )(hn, num, den, h, w, b)


def _final_body(h_ref, w_ref, b_ref, o_ref):
    z = jnp.dot(h_ref[...], w_ref[...],
                preferred_element_type=jnp.float32) + b_ref[...]
    col = lax.broadcasted_iota(jnp.int32, z.shape, 1)
    valid = col < OUT_DIM
    zm = jnp.where(valid, z, -jnp.inf)
    mx = jnp.max(zm, axis=1, keepdims=True)
    e = jnp.where(valid, jnp.exp(zm - mx), 0.0)
    lse = jnp.log(jnp.sum(e, axis=1, keepdims=True))
    out = zm - mx - lse
    o_ref[...] = out[:, :OUT_DIM]


def _final(h, wp, bp):
    return pl.pallas_call(
        _final_body,
        grid=(N // RB,),
        in_specs=[pl.BlockSpec((RB, HID), lambda i: (i, 0)),
                  pl.BlockSpec((HID, HID), lambda i: (0, 0)),
                  pl.BlockSpec((1, HID), lambda i: (0, 0))],
        out_specs=pl.BlockSpec((RB, OUT_DIM), lambda i: (i, 0)),
        out_shape=jax.ShapeDtypeStruct((N, OUT_DIM), jnp.float32),
    )(h, wp, bp)


def kernel(node_feats, edge_index, W_enc, b_enc, W_conv, b_conv,
           bn_gamma, bn_beta, W_out, b_out):
    src = edge_index[0]
    dst = edge_index[1]
    pad = E_PAD - E
    src_p = jnp.concatenate([src, jnp.zeros((pad,), jnp.int32)])
    dst_p = jnp.concatenate([dst, jnp.full((pad,), N, jnp.int32)])
    zeros_blk = jnp.zeros((ZR, HID), jnp.float32)

    h = _encoder(node_feats, W_enc, b_enc.reshape(1, HID))
    for i in range(NUM_LAYERS):
        hn, tg, tp = _pre(h, bn_gamma[i].reshape(1, HID),
                          bn_beta[i].reshape(1, HID))
        den, num = _sc_aggregate(tg, tp, src_p, dst_p, zeros_blk)
        h = _post(hn, num, den, h, W_conv[i], b_conv[i].reshape(1, HID))

    wp = jnp.pad(W_out, ((0, 0), (0, HID - OUT_DIM)))
    bp = jnp.pad(b_out, (0, HID - OUT_DIM)).reshape(1, HID)
    return _final(h, wp, bp)


# speedup vs baseline: 11.1732x; 1.3512x over previous
import functools

import jax
import jax.numpy as jnp
from jax import lax
from jax.experimental import pallas as pl
from jax.experimental.pallas import tpu as pltpu
from jax.experimental.pallas import tpu_sc as plsc

N = 10000
E = 320000
HID = 128
OUT_DIM = 40
NUM_LAYERS = 7
EPS = 1e-7
BN_EPS = 1e-5

NT = 16
EB = 128
NB = 2 * (-(-E // (NT * EB * 2)))
E_PAD = NT * NB * EB
ACC_ROWS = 10240
ZR = ACC_ROWS // NT
RB = 2000


def _sc_aggregate(tg, tp, src_p, dst_p, zeros_blk):
    mesh = plsc.VectorSubcoreMesh(core_axis_name="c", subcore_axis_name="s")

    @functools.partial(
        pl.kernel,
        out_type=(jax.ShapeDtypeStruct((ACC_ROWS, HID), jnp.float32),
                  jax.ShapeDtypeStruct((ACC_ROWS, HID), jnp.float32)),
        mesh=mesh,
        scratch_types=[
            pltpu.VMEM((EB,), jnp.int32),
            pltpu.VMEM((EB,), jnp.int32),
            pltpu.VMEM((EB,), jnp.int32),
            pltpu.VMEM((EB,), jnp.int32),
            pltpu.VMEM((EB, HID), jnp.float32),
            pltpu.VMEM((EB, HID), jnp.float32),
            pltpu.VMEM_SHARED((ACC_ROWS, HID), jnp.float32),
        ] + [pltpu.SemaphoreType.DMA] * 8,
    )
    def k(tg_hbm, tp_hbm, src_hbm, dst_hbm, z_hbm, den_out, num_out,
          sidx0, sidx1, didx0, didx1, rows0, rows1, acc,
          isS0, isS1, isD0, isD1, gsem0, gsem1, ssem0, ssem1):
        c = lax.axis_index("c")
        s = lax.axis_index("s")

        pltpu.sync_copy(z_hbm, acc.at[pl.ds(s * ZR, ZR)])
        plsc.subcore_barrier()

        def run(t_hbm):
            def slot(b, si, di, rows, isS, isD, gsem, ssem):
                pltpu.make_async_copy(t_hbm.at[si], rows, gsem).wait()

                @pl.when(b + 2 < NB)
                def _():
                    pltpu.async_copy(src_hbm.at[s, b + 2], si, isS)

                pltpu.make_async_copy(dst_hbm.at[s, b], di, isD).wait()
                pltpu.async_copy(rows, acc.at[di], ssem, add=True).wait()

                @pl.when(b + 2 < NB)
                def _():
                    pltpu.async_copy(dst_hbm.at[s, b + 2], di, isD)
                    pltpu.make_async_copy(src_hbm.at[s, b + 2], si, isS).wait()
                    pltpu.async_copy(t_hbm.at[si], rows, gsem)

            pltpu.async_copy(dst_hbm.at[s, 0], didx0, isD0)
            pltpu.async_copy(dst_hbm.at[s, 1], didx1, isD1)
            pltpu.sync_copy(src_hbm.at[s, 0], sidx0)
            pltpu.sync_copy(src_hbm.at[s, 1], sidx1)
            pltpu.async_copy(t_hbm.at[sidx0], rows0, gsem0)
            pltpu.async_copy(t_hbm.at[sidx1], rows1, gsem1)

            def body(j2, carry):
                b0 = 2 * j2
                slot(b0, sidx0, didx0, rows0, isS0, isD0, gsem0, ssem0)
                slot(b0 + 1, sidx1, didx1, rows1, isS1, isD1, gsem1, ssem1)
                return carry

            lax.fori_loop(0, NB // 2, body, 0)

        @pl.when(c == 0)
        def _():
            run(tg_hbm)

        @pl.when(c == 1)
        def _():
            run(tp_hbm)

        plsc.subcore_barrier()

        @pl.when(c == 0)
        def _():
            pltpu.sync_copy(acc.at[pl.ds(s * ZR, ZR)],
                            den_out.at[pl.ds(s * ZR, ZR)])

        @pl.when(c == 1)
        def _():
            pltpu.sync_copy(acc.at[pl.ds(s * ZR, ZR)],
                            num_out.at[pl.ds(s * ZR, ZR)])

    den, num = k(tg, tp, src_p, dst_p, zeros_blk)
    return den[:N], num[:N]


def _enc_body(x_ref, w_ref, b_ref, o_ref):
    o_ref[...] = jnp.dot(x_ref[...], w_ref[...],
                         preferred_element_type=jnp.float32) + b_ref[...]


def _encoder(x, w, b):
    return pl.pallas_call(
        _enc_body,
        grid=(N // RB,),
        in_specs=[pl.BlockSpec((RB, HID), lambda i: (i, 0)),
                  pl.BlockSpec((HID, HID), lambda i: (0, 0)),
                  pl.BlockSpec((1, HID), lambda i: (0, 0))],
        out_specs=pl.BlockSpec((RB, HID), lambda i: (i, 0)),
        out_shape=jax.ShapeDtypeStruct((N, HID), jnp.float32),
    )(x, w, b)


def _pre_body(h_ref, g_ref, b_ref, hn_ref, tg_ref, tp_ref):
    h = h_ref[...]
    mean = jnp.mean(h, axis=0, keepdims=True)
    d = h - mean
    var = jnp.mean(d * d, axis=0, keepdims=True)
    hn = d * lax.rsqrt(var + BN_EPS) * g_ref[...] + b_ref[...]
    hn = jnp.maximum(hn, 0.0)
    g = hn + EPS
    cmax = jnp.max(g, axis=0, keepdims=True)
    tg = jnp.exp(g - cmax)
    hn_ref[...] = hn
    tg_ref[...] = tg
    tp_ref[...] = g * tg


def _pre(h, gamma, beta):
    spec = pl.BlockSpec((N, HID), lambda: (0, 0))
    vspec = pl.BlockSpec((1, HID), lambda: (0, 0))
    return pl.pallas_call(
        _pre_body,
        in_specs=[spec, vspec, vspec],
        out_specs=(spec, spec, spec),
        out_shape=(jax.ShapeDtypeStruct((N, HID), jnp.float32),) * 3,
    )(h, gamma, beta)


def _post_body(hn_ref, num_ref, den_ref, h_ref, w_ref, b_ref, o_ref):
    msg = num_ref[...] / (den_ref[...] + 1e-16)
    l2 = jnp.sqrt(jnp.sum(msg * msg, axis=1, keepdims=True))
    hn = hn_ref[...]
    fn = jnp.sqrt(jnp.sum(hn * hn, axis=1, keepdims=True))
    msg = msg / jnp.maximum(l2, 1e-12) * fn
    feats = hn + msg
    o_ref[...] = h_ref[...] + jnp.dot(feats, w_ref[...],
                                      preferred_element_type=jnp.float32) + b_ref[...]


def _post(hn, num, den, h, w, b):
    rspec = pl.BlockSpec((RB, HID), lambda i: (i, 0))
    return pl.pallas_call(
        _post_body,
        grid=(N // RB,),
        in_specs=[rspec, rspec, rspec, rspec,
                  pl.BlockSpec((HID, HID), lambda i: (0, 0)),
                  pl.BlockSpec((1, HID), lambda i: (0, 0))],
        out_specs=rspec,
        out_shape=jax.ShapeDtypeStruct((N, HID), jnp.float32),
    )(hn, num, den, h, w, b)


def _final_body(h_ref, w_ref, b_ref, o_ref):
    z = jnp.dot(h_ref[...], w_ref[...],
                preferred_element_type=jnp.float32) + b_ref[...]
    col = lax.broadcasted_iota(jnp.int32, z.shape, 1)
    valid = col < OUT_DIM
    zm = jnp.where(valid, z, -jnp.inf)
    mx = jnp.max(zm, axis=1, keepdims=True)
    e = jnp.where(valid, jnp.exp(zm - mx), 0.0)
    lse = jnp.log(jnp.sum(e, axis=1, keepdims=True))
    out = zm - mx - lse
    o_ref[...] = out[:, :OUT_DIM]


def _final(h, wp, bp):
    return pl.pallas_call(
        _final_body,
        grid=(N // RB,),
        in_specs=[pl.BlockSpec((RB, HID), lambda i: (i, 0)),
                  pl.BlockSpec((HID, HID), lambda i: (0, 0)),
                  pl.BlockSpec((1, HID), lambda i: (0, 0))],
        out_specs=pl.BlockSpec((RB, OUT_DIM), lambda i: (i, 0)),
        out_shape=jax.ShapeDtypeStruct((N, OUT_DIM), jnp.float32),
    )(h, wp, bp)


def kernel(node_feats, edge_index, W_enc, b_enc, W_conv, b_conv,
           bn_gamma, bn_beta, W_out, b_out):
    src = edge_index[0]
    dst = edge_index[1]
    pad = E_PAD - E
    src_p = jnp.concatenate([src, jnp.zeros((pad,), jnp.int32)]).reshape(
        NT, NB, EB)
    dst_p = jnp.concatenate([dst, jnp.full((pad,), N, jnp.int32)]).reshape(
        NT, NB, EB)
    zeros_blk = jnp.zeros((ZR, HID), jnp.float32)

    h = _encoder(node_feats, W_enc, b_enc.reshape(1, HID))
    for i in range(NUM_LAYERS):
        hn, tg, tp = _pre(h, bn_gamma[i].reshape(1, HID),
                          bn_beta[i].reshape(1, HID))
        den, num = _sc_aggregate(tg, tp, src_p, dst_p, zeros_blk)
        h = _post(hn, num, den, h, W_conv[i], b_conv[i].reshape(1, HID))

    wp = jnp.pad(W_out, ((0, 0), (0, HID - OUT_DIM)))
    bp = jnp.pad(b_out, (0, HID - OUT_DIM)).reshape(1, HID)
    return _final(h, wp, bp)


# speedup vs baseline: 13.4259x; 1.2016x over previous
import functools

import jax
import jax.numpy as jnp
from jax import lax
from jax.experimental import pallas as pl
from jax.experimental.pallas import tpu as pltpu
from jax.experimental.pallas import tpu_sc as plsc

N = 10000
E = 320000
HID = 128
OUT_DIM = 40
NUM_LAYERS = 7
EPS = 1e-7
BN_EPS = 1e-5

NT = 16
EB = 120
NB = 3 * (-(-E // (NT * EB * 3)))
E_PAD = NT * NB * EB
ACC_ROWS = 10240
ZR = ACC_ROWS // NT
RB = 2000


def _sc_aggregate(tg, tp, src_p, dst_p, zeros_blk):
    mesh = plsc.VectorSubcoreMesh(core_axis_name="c", subcore_axis_name="s")

    @functools.partial(
        pl.kernel,
        out_type=(jax.ShapeDtypeStruct((ACC_ROWS, HID), jnp.float32),
                  jax.ShapeDtypeStruct((ACC_ROWS, HID), jnp.float32)),
        mesh=mesh,
        scratch_types=(
            [pltpu.VMEM((EB,), jnp.int32)] * 6
            + [pltpu.VMEM((EB, HID), jnp.float32)] * 3
            + [pltpu.VMEM_SHARED((ACC_ROWS, HID), jnp.float32)]
            + [pltpu.SemaphoreType.DMA] * 12
        ),
    )
    def k(tg_hbm, tp_hbm, src_hbm, dst_hbm, z_hbm, den_out, num_out,
          sidx0, sidx1, sidx2, didx0, didx1, didx2, rows0, rows1, rows2, acc,
          isS0, isS1, isS2, isD0, isD1, isD2,
          gsem0, gsem1, gsem2, ssem0, ssem1, ssem2):
        c = lax.axis_index("c")
        s = lax.axis_index("s")

        SI = [sidx0, sidx1, sidx2]
        DI = [didx0, didx1, didx2]
        RW = [rows0, rows1, rows2]
        IS = [isS0, isS1, isS2]
        ID = [isD0, isD1, isD2]
        GS = [gsem0, gsem1, gsem2]
        SS = [ssem0, ssem1, ssem2]

        pltpu.sync_copy(z_hbm, acc.at[pl.ds(s * ZR, ZR)])
        plsc.subcore_barrier()

        def run(t_hbm):
            def sidx_fetch(b, r):
                pltpu.async_copy(src_hbm.at[s, b], SI[r], IS[r])

            def sidx_wait(b, r):
                pltpu.make_async_copy(src_hbm.at[s, b], SI[r], IS[r]).wait()

            def didx_fetch(b, r):
                pltpu.async_copy(dst_hbm.at[s, b], DI[r], ID[r])

            def didx_wait(b, r):
                pltpu.make_async_copy(dst_hbm.at[s, b], DI[r], ID[r]).wait()

            def gather_start(r):
                pltpu.async_copy(t_hbm.at[SI[r]], RW[r], GS[r])

            def gather_wait(r):
                pltpu.make_async_copy(t_hbm.at[SI[r]], RW[r], GS[r]).wait()

            def scatter_start(r):
                pltpu.async_copy(RW[r], acc.at[DI[r]], SS[r], add=True)

            def scatter_wait(r):
                pltpu.make_async_copy(RW[r], acc.at[DI[r]], SS[r]).wait()

            didx_fetch(0, 0)
            didx_fetch(1, 1)
            pltpu.sync_copy(src_hbm.at[s, 0], sidx0)
            pltpu.sync_copy(src_hbm.at[s, 1], sidx1)
            gather_start(0)
            gather_start(1)
            sidx_fetch(2, 2)

            def visit(b, r, v):
                rp = (r + 2) % 3
                gather_wait(r)

                @pl.when(b + 3 < NB)
                def _():
                    sidx_fetch(b + 3, r)

                if v == 0:
                    @pl.when(b >= 1)
                    def _():
                        scatter_wait(rp)
                else:
                    scatter_wait(rp)

                @pl.when(b + 2 < NB)
                def _():
                    didx_fetch(b + 2, rp)

                didx_wait(b, r)
                scatter_start(r)

                @pl.when(b + 2 < NB)
                def _():
                    sidx_wait(b + 2, rp)
                    gather_start(rp)

            def body(j, carry):
                b0 = 3 * j
                visit(b0, 0, 0)
                visit(b0 + 1, 1, 1)
                visit(b0 + 2, 2, 2)
                return carry

            lax.fori_loop(0, NB // 3, body, 0)
            scatter_wait((NB - 1) % 3)

        @pl.when(c == 0)
        def _():
            run(tg_hbm)

        @pl.when(c == 1)
        def _():
            run(tp_hbm)

        plsc.subcore_barrier()

        @pl.when(c == 0)
        def _():
            pltpu.sync_copy(acc.at[pl.ds(s * ZR, ZR)],
                            den_out.at[pl.ds(s * ZR, ZR)])

        @pl.when(c == 1)
        def _():
            pltpu.sync_copy(acc.at[pl.ds(s * ZR, ZR)],
                            num_out.at[pl.ds(s * ZR, ZR)])

    den, num = k(tg, tp, src_p, dst_p, zeros_blk)
    return den[:N], num[:N]


def _enc_body(x_ref, w_ref, b_ref, o_ref):
    o_ref[...] = jnp.dot(x_ref[...], w_ref[...],
                         preferred_element_type=jnp.float32) + b_ref[...]


def _encoder(x, w, b):
    return pl.pallas_call(
        _enc_body,
        grid=(N // RB,),
        in_specs=[pl.BlockSpec((RB, HID), lambda i: (i, 0)),
                  pl.BlockSpec((HID, HID), lambda i: (0, 0)),
                  pl.BlockSpec((1, HID), lambda i: (0, 0))],
        out_specs=pl.BlockSpec((RB, HID), lambda i: (i, 0)),
        out_shape=jax.ShapeDtypeStruct((N, HID), jnp.float32),
    )(x, w, b)


def _pre_body(h_ref, g_ref, b_ref, hn_ref, tg_ref, tp_ref):
    h = h_ref[...]
    mean = jnp.mean(h, axis=0, keepdims=True)
    d = h - mean
    var = jnp.mean(d * d, axis=0, keepdims=True)
    hn = d * lax.rsqrt(var + BN_EPS) * g_ref[...] + b_ref[...]
    hn = jnp.maximum(hn, 0.0)
    g = hn + EPS
    cmax = jnp.max(g, axis=0, keepdims=True)
    tg = jnp.exp(g - cmax)
    hn_ref[...] = hn
    tg_ref[...] = tg
    tp_ref[...] = g * tg


def _pre(h, gamma, beta):
    spec = pl.BlockSpec((N, HID), lambda: (0, 0))
    vspec = pl.BlockSpec((1, HID), lambda: (0, 0))
    return pl.pallas_call(
        _pre_body,
        in_specs=[spec, vspec, vspec],
        out_specs=(spec, spec, spec),
        out_shape=(jax.ShapeDtypeStruct((N, HID), jnp.float32),) * 3,
    )(h, gamma, beta)


def _post_body(hn_ref, num_ref, den_ref, h_ref, w_ref, b_ref, o_ref):
    msg = num_ref[...] / (den_ref[...] + 1e-16)
    l2 = jnp.sqrt(jnp.sum(msg * msg, axis=1, keepdims=True))
    hn = hn_ref[...]
    fn = jnp.sqrt(jnp.sum(hn * hn, axis=1, keepdims=True))
    msg = msg / jnp.maximum(l2, 1e-12) * fn
    feats = hn + msg
    o_ref[...] = h_ref[...] + jnp.dot(feats, w_ref[...],
                                      preferred_element_type=jnp.float32) + b_ref[...]


def _post(hn, num, den, h, w, b):
    rspec = pl.BlockSpec((RB, HID), lambda i: (i, 0))
    return pl.pallas_call(
        _post_body,
        grid=(N // RB,),
        in_specs=[rspec, rspec, rspec, rspec,
                  pl.BlockSpec((HID, HID), lambda i: (0, 0)),
                  pl.BlockSpec((1, HID), lambda i: (0, 0))],
        out_specs=rspec,
        out_shape=jax.ShapeDtypeStruct((N, HID), jnp.float32),
    )(hn, num, den, h, w, b)


def _final_body(h_ref, w_ref, b_ref, o_ref):
    z = jnp.dot(h_ref[...], w_ref[...],
                preferred_element_type=jnp.float32) + b_ref[...]
    col = lax.broadcasted_iota(jnp.int32, z.shape, 1)
    valid = col < OUT_DIM
    zm = jnp.where(valid, z, -jnp.inf)
    mx = jnp.max(zm, axis=1, keepdims=True)
    e = jnp.where(valid, jnp.exp(zm - mx), 0.0)
    lse = jnp.log(jnp.sum(e, axis=1, keepdims=True))
    out = zm - mx - lse
    o_ref[...] = out[:, :OUT_DIM]


def _final(h, wp, bp):
    return pl.pallas_call(
        _final_body,
        grid=(N // RB,),
        in_specs=[pl.BlockSpec((RB, HID), lambda i: (i, 0)),
                  pl.BlockSpec((HID, HID), lambda i: (0, 0)),
                  pl.BlockSpec((1, HID), lambda i: (0, 0))],
        out_specs=pl.BlockSpec((RB, OUT_DIM), lambda i: (i, 0)),
        out_shape=jax.ShapeDtypeStruct((N, OUT_DIM), jnp.float32),
    )(h, wp, bp)


def kernel(node_feats, edge_index, W_enc, b_enc, W_conv, b_conv,
           bn_gamma, bn_beta, W_out, b_out):
    src = edge_index[0]
    dst = edge_index[1]
    pad = E_PAD - E
    src_p = jnp.concatenate([src, jnp.zeros((pad,), jnp.int32)]).reshape(
        NT, NB, EB)
    dst_p = jnp.concatenate([dst, jnp.full((pad,), N, jnp.int32)]).reshape(
        NT, NB, EB)
    zeros_blk = jnp.zeros((ZR, HID), jnp.float32)

    h = _encoder(node_feats, W_enc, b_enc.reshape(1, HID))
    for i in range(NUM_LAYERS):
        hn, tg, tp = _pre(h, bn_gamma[i].reshape(1, HID),
                          bn_beta[i].reshape(1, HID))
        den, num = _sc_aggregate(tg, tp, src_p, dst_p, zeros_blk)
        h = _post(hn, num, den, h, W_conv[i], b_conv[i].reshape(1, HID))

    wp = jnp.pad(W_out, ((0, 0), (0, HID - OUT_DIM)))
    bp = jnp.pad(b_out, (0, HID - OUT_DIM)).reshape(1, HID)
    return _final(h, wp, bp)
